# baseline (speedup 1.0000x reference)
import jax
import jax.numpy as jnp
from jax.experimental import pallas as pl
from jax.experimental.pallas import tpu as pltpu

E = 8
K = 2
CHUNK = 1024
NBUF = 12


def _gate_block(hid_hbm, w_ref, idx_ref, wgt_ref, buf, sems):
    i = pl.program_id(0)
    nsteps = pl.num_programs(0)

    @pl.when(i == 0)
    def _warmup():
        for s in range(NBUF):
            pltpu.make_async_copy(
                hid_hbm.at[pl.ds(s * CHUNK, CHUNK), :],
                buf.at[s], sems.at[s]).start(priority=s % 2)

    slot = jax.lax.rem(i, NBUF)
    for s in range(NBUF):
        @pl.when(slot == s)
        def _wait(s=s):
            pltpu.make_async_copy(
                hid_hbm.at[pl.ds(i * CHUNK, CHUNK), :],
                buf.at[s], sems.at[s]).wait()

    h = buf[slot]
    w = w_ref[...]
    logits = jax.lax.dot_general(
        w, h, (((1,), (1,)), ((), ())),
        preferred_element_type=jnp.float32)

    sub = jax.lax.broadcasted_iota(jnp.int32, logits.shape, 0)
    m1 = jnp.max(logits, axis=0, keepdims=True)
    i1 = jnp.min(jnp.where(logits >= m1, sub, E), axis=0, keepdims=True)
    masked = jnp.where(sub == i1, -jnp.inf, logits)
    m2 = jnp.max(masked, axis=0, keepdims=True)
    i2 = jnp.min(jnp.where(masked >= m2, sub, E), axis=0, keepdims=True)

    e2 = jnp.exp(m2 - m1)
    w1 = 1.0 / (1.0 + e2)
    idx_ref[...] = jnp.concatenate([i1, i2], axis=0)
    wgt_ref[...] = jnp.concatenate([w1, 1.0 - w1], axis=0)

    nxt = i + NBUF
    for s in range(NBUF):
        @pl.when(jnp.logical_and(nxt < nsteps, slot == s))
        def _prefetch(s=s):
            pltpu.make_async_copy(
                hid_hbm.at[pl.ds(nxt * CHUNK, CHUNK), :],
                buf.at[s], sems.at[s]).start(priority=s % 2)


@jax.jit
def kernel(hidden, weight):
    n, d = hidden.shape
    grid = (n // CHUNK,)
    idx, wgt = pl.pallas_call(
        _gate_block,
        grid=grid,
        in_specs=[
            pl.BlockSpec(memory_space=pltpu.MemorySpace.HBM),
            pl.BlockSpec((E, d), lambda i: (0, 0)),
        ],
        out_specs=[
            pl.BlockSpec((K, CHUNK), lambda i: (0, i)),
            pl.BlockSpec((K, CHUNK), lambda i: (0, i)),
        ],
        out_shape=[
            jax.ShapeDtypeStruct((K, n), jnp.int32),
            jax.ShapeDtypeStruct((K, n), jnp.float32),
        ],
        scratch_shapes=[
            pltpu.VMEM((NBUF, CHUNK, d), jnp.float32),
            pltpu.SemaphoreType.DMA((NBUF,)),
        ],
        compiler_params=pltpu.CompilerParams(
            dimension_semantics=("arbitrary",)),
    )(hidden, weight)
    return idx.T, wgt.T


# speedup vs baseline: 1.0489x; 1.0489x over previous
import jax
import jax.numpy as jnp
from jax.experimental import pallas as pl
from jax.experimental.pallas import tpu as pltpu

E = 8
K = 2
CHUNK = 2048
NBUF = 6


def _gate_block(hid_hbm, w_ref, idx_ref, wgt_ref, buf, sems):
    i = pl.program_id(0)
    nsteps = pl.num_programs(0)

    @pl.when(i == 0)
    def _warmup():
        for s in range(NBUF):
            pltpu.make_async_copy(
                hid_hbm.at[pl.ds(s * CHUNK, CHUNK), :],
                buf.at[s], sems.at[s]).start(priority=s % 2)

    slot = jax.lax.rem(i, NBUF)
    for s in range(NBUF):
        @pl.when(slot == s)
        def _wait(s=s):
            pltpu.make_async_copy(
                hid_hbm.at[pl.ds(i * CHUNK, CHUNK), :],
                buf.at[s], sems.at[s]).wait()

    h = buf[slot]
    w = w_ref[...]
    logits = jax.lax.dot_general(
        w, h, (((1,), (1,)), ((), ())),
        preferred_element_type=jnp.float32)

    sub = jax.lax.broadcasted_iota(jnp.int32, logits.shape, 0)
    m1 = jnp.max(logits, axis=0, keepdims=True)
    i1 = jnp.min(jnp.where(logits >= m1, sub, E), axis=0, keepdims=True)
    masked = jnp.where(sub == i1, -jnp.inf, logits)
    m2 = jnp.max(masked, axis=0, keepdims=True)
    i2 = jnp.min(jnp.where(masked >= m2, sub, E), axis=0, keepdims=True)

    e2 = jnp.exp(m2 - m1)
    w1 = 1.0 / (1.0 + e2)
    idx_ref[...] = jnp.concatenate([i1, i2], axis=0)
    wgt_ref[...] = jnp.concatenate([w1, 1.0 - w1], axis=0)

    nxt = i + NBUF
    for s in range(NBUF):
        @pl.when(jnp.logical_and(nxt < nsteps, slot == s))
        def _prefetch(s=s):
            pltpu.make_async_copy(
                hid_hbm.at[pl.ds(nxt * CHUNK, CHUNK), :],
                buf.at[s], sems.at[s]).start(priority=s % 2)


@jax.jit
def kernel(hidden, weight):
    n, d = hidden.shape
    grid = (n // CHUNK,)
    idx, wgt = pl.pallas_call(
        _gate_block,
        grid=grid,
        in_specs=[
            pl.BlockSpec(memory_space=pltpu.MemorySpace.HBM),
            pl.BlockSpec((E, d), lambda i: (0, 0)),
        ],
        out_specs=[
            pl.BlockSpec((K, CHUNK), lambda i: (0, i)),
            pl.BlockSpec((K, CHUNK), lambda i: (0, i)),
        ],
        out_shape=[
            jax.ShapeDtypeStruct((K, n), jnp.int32),
            jax.ShapeDtypeStruct((K, n), jnp.float32),
        ],
        scratch_shapes=[
            pltpu.VMEM((NBUF, CHUNK, d), jnp.float32),
            pltpu.SemaphoreType.DMA((NBUF,)),
        ],
        compiler_params=pltpu.CompilerParams(
            dimension_semantics=("arbitrary",)),
    )(hidden, weight)
    return idx.T, wgt.T
